# Initial kernel scaffold; baseline (speedup 1.0000x reference)
import jax
import jax.numpy as jnp
from jax.experimental import pallas as pl


def kernel(x, edge_index, W, b_w, a_w, a_b):
    raise NotImplementedError("write your pallas kernel here")


# speedup vs baseline: 6.2630x; 6.2630x over previous
import functools

import jax
import jax.numpy as jnp
from jax import lax
from jax.experimental import pallas as pl
from jax.experimental.pallas import tpu as pltpu
from jax.experimental.pallas import tpu_sc as plsc

N = 10000
D = 128
E = 320000

NC = 2
NS = 16
NW = NC * NS
EPW = E // NW
K = 80
NCHUNK = EPW // K
NPAD = 10240
RPT = NPAD // NS
ZR = 80


def _lin_body(x_ref, u_ref, b_ref, z_ref):
    z_ref[...] = (
        jnp.dot(x_ref[...], u_ref[...], preferred_element_type=jnp.float32)
        + b_ref[...]
    )


def _combine_body(p0_ref, p1_ref, z_ref, x_ref, o_ref):
    z = z_ref[..., 0:1] + z_ref[..., 1:2]
    att = jnp.maximum(z, 0.2 * z)
    o_ref[...] = p0_ref[...] + p1_ref[...] + att * x_ref[...]


def _sc_edge_kernel(
    x_hbm, src_hbm, obj_hbm, s1_hbm, s2_hbm, out_hbm,
    s1_v, s2_v, sidx_v, oidx_v, rows_v, att_v, zero_v, acc_sh, sem,
):
    cid = lax.axis_index("c")
    sid = lax.axis_index("s")
    wid = cid * NS + sid

    pltpu.sync_copy(s1_hbm, s1_v)
    pltpu.sync_copy(s2_hbm, s2_v)

    def zfill(i, _):
        for j in range(D // 16):
            zero_v[i, pl.ds(j * 16, 16)] = jnp.zeros((16,), jnp.float32)
        return 0

    lax.fori_loop(0, ZR, zfill, 0)
    for r in range(RPT // ZR):
        pltpu.sync_copy(zero_v, acc_sh.at[pl.ds(sid * RPT + r * ZR, ZR)])
    plsc.subcore_barrier()

    base = wid * EPW

    def chunk(c, _):
        eb = base + c * K
        pltpu.sync_copy(src_hbm.at[pl.ds(eb, K)], sidx_v)
        pltpu.sync_copy(obj_hbm.at[pl.ds(eb, K)], oidx_v)
        pltpu.async_copy(x_hbm.at[oidx_v], rows_v, sem).wait()
        for i in range(K // 16):
            si = sidx_v[pl.ds(i * 16, 16)]
            oi = oidx_v[pl.ds(i * 16, 16)]
            z = plsc.load_gather(s1_v, [si]) + plsc.load_gather(s2_v, [oi])
            att_v[pl.ds(i * 16, 16)] = jnp.maximum(z, 0.2 * z)

        def scale(k, _):
            a = plsc.load_gather(att_v, [lax.broadcast(k, (16,))])
            for j in range(D // 16):
                rows_v[k, pl.ds(j * 16, 16)] = rows_v[k, pl.ds(j * 16, 16)] * a
            return 0

        lax.fori_loop(0, K, scale, 0)
        pltpu.sync_copy(rows_v, acc_sh.at[sidx_v], add=True)
        return 0

    lax.fori_loop(0, NCHUNK, chunk, 0)

    plsc.subcore_barrier()
    pltpu.sync_copy(
        acc_sh.at[pl.ds(sid * RPT, RPT)],
        out_hbm.at[cid, pl.ds(sid * RPT, RPT)],
    )


@jax.jit
def kernel(x, edge_index, W, b_w, a_w, a_b):
    a1 = a_w[0, :D]
    a2 = a_w[0, D:]
    u = jnp.zeros((D, 8), jnp.float32).at[:, 0].set(W.T @ a1).at[:, 1].set(W.T @ a2)
    bias = (
        jnp.zeros((1, 8), jnp.float32)
        .at[0, 0].set(jnp.dot(b_w, a1) + a_b[0])
        .at[0, 1].set(jnp.dot(b_w, a2))
    )

    zb = 1000
    z = pl.pallas_call(
        _lin_body,
        grid=(N // zb,),
        in_specs=[
            pl.BlockSpec((zb, D), lambda i: (i, 0)),
            pl.BlockSpec((D, 8), lambda i: (0, 0)),
            pl.BlockSpec((1, 8), lambda i: (0, 0)),
        ],
        out_specs=pl.BlockSpec((zb, 8), lambda i: (i, 0)),
        out_shape=jax.ShapeDtypeStruct((N, 8), jnp.float32),
    )(x, u, bias)

    s1 = z[:, 0]
    s2 = z[:, 1]

    mesh = plsc.VectorSubcoreMesh(core_axis_name="c", subcore_axis_name="s")
    sc_edge = pl.kernel(
        _sc_edge_kernel,
        mesh=mesh,
        compiler_params=pltpu.CompilerParams(needs_layout_passes=False),
        out_type=jax.ShapeDtypeStruct((NC, NPAD, D), jnp.float32),
        scratch_types=[
            pltpu.VMEM((N,), jnp.float32),
            pltpu.VMEM((N,), jnp.float32),
            pltpu.VMEM((K,), jnp.int32),
            pltpu.VMEM((K,), jnp.int32),
            pltpu.VMEM((K, D), jnp.float32),
            pltpu.VMEM((K,), jnp.float32),
            pltpu.VMEM((ZR, D), jnp.float32),
            pltpu.VMEM_SHARED((NPAD, D), jnp.float32),
            pltpu.SemaphoreType.DMA,
        ],
    )
    partial_acc = sc_edge(x, edge_index[0], edge_index[1], s1, s2)

    cb = 1000
    out = pl.pallas_call(
        _combine_body,
        grid=(N // cb,),
        in_specs=[
            pl.BlockSpec((cb, D), lambda i: (i, 0)),
            pl.BlockSpec((cb, D), lambda i: (i, 0)),
            pl.BlockSpec((cb, 8), lambda i: (i, 0)),
            pl.BlockSpec((cb, D), lambda i: (i, 0)),
        ],
        out_specs=pl.BlockSpec((cb, D), lambda i: (i, 0)),
        out_shape=jax.ShapeDtypeStruct((N, D), jnp.float32),
    )(partial_acc[0], partial_acc[1], z, x)
    return out


# speedup vs baseline: 7.5004x; 1.1976x over previous
import jax
import jax.numpy as jnp
from jax import lax
from jax.experimental import pallas as pl
from jax.experimental.pallas import tpu as pltpu
from jax.experimental.pallas import tpu_sc as plsc

N = 10000
D = 128
E = 320000

NC = 2
NS = 16
NW = NC * NS
K = 48
NCHUNK = 209
EPW = NCHUNK * K
EPAD = NW * EPW
NBUF = 4
NPAD = 10112
RPT = NPAD // NS
TBL = 10016


def _lin_body(x_ref, u_ref, b_ref, z_ref):
    z_ref[...] = (
        jnp.dot(x_ref[...], u_ref[...], preferred_element_type=jnp.float32)
        + b_ref[...]
    )


def _combine_body(p0_ref, p1_ref, z_ref, x_ref, o_ref):
    z = z_ref[..., 0:1] + z_ref[..., 1:2]
    att = jnp.maximum(z, 0.2 * z)
    o_ref[...] = p0_ref[...] + p1_ref[...] + att * x_ref[...]


def _sc_edge_kernel(
    x_hbm, src_hbm, obj_hbm, s1_hbm, s2_hbm, out_hbm,
    s1_v, s2_v, sidx_v, oidx_v, rows_v, att_v, acc_sh,
    g0, g1, g2, g3, c0, c1, c2, c3, i0, i1, i2, i3,
):
    gsems = [g0, g1, g2, g3]
    ssems = [c0, c1, c2, c3]
    isems = [i0, i1, i2, i3]
    cid = lax.axis_index("c")
    sid = lax.axis_index("s")
    wid = cid * NS + sid
    ebase = wid * EPW

    pltpu.sync_copy(s1_hbm, s1_v.at[pl.ds(0, N)])
    pltpu.sync_copy(s2_hbm, s2_v.at[pl.ds(0, N)])

    def zfill(i, _):
        for j in range(D // 16):
            rows_v[0, i, pl.ds(j * 16, 16)] = jnp.zeros((16,), jnp.float32)
        return 0

    lax.fori_loop(0, K, zfill, 0)
    for r in range(13):
        pltpu.sync_copy(rows_v.at[0], acc_sh.at[pl.ds(sid * RPT + r * K, K)])
    pltpu.sync_copy(
        rows_v.at[0].at[pl.ds(0, 8)], acc_sh.at[pl.ds(sid * RPT + 624, 8)]
    )
    plsc.subcore_barrier()

    def start_idx(c, b):
        pltpu.async_copy(
            src_hbm.at[pl.ds(ebase + c * K, K)], sidx_v.at[b], isems[b]
        )
        pltpu.async_copy(
            obj_hbm.at[pl.ds(ebase + c * K, K)], oidx_v.at[b], isems[b]
        )

    def wait_idx(b):
        pltpu.make_async_copy(
            src_hbm.at[pl.ds(0, K)], sidx_v.at[b], isems[b]
        ).wait()
        pltpu.make_async_copy(
            obj_hbm.at[pl.ds(0, K)], oidx_v.at[b], isems[b]
        ).wait()

    def start_gather(b):
        pltpu.async_copy(x_hbm.at[oidx_v.at[b]], rows_v.at[b], gsems[b])

    def wait_gather(b):
        pltpu.make_async_copy(
            x_hbm.at[oidx_v.at[b]], rows_v.at[b], gsems[b]
        ).wait()

    def start_scatter(b):
        pltpu.async_copy(
            rows_v.at[b], acc_sh.at[sidx_v.at[b]], ssems[b], add=True
        )

    def drain_scatter(b):
        pltpu.make_async_copy(
            rows_v.at[b], acc_sh.at[sidx_v.at[b]], ssems[b]
        ).wait()

    def process(b):
        for t in range(K // 16):
            si = sidx_v[b, pl.ds(t * 16, 16)]
            oi = oidx_v[b, pl.ds(t * 16, 16)]
            z = plsc.load_gather(s1_v, [si]) + plsc.load_gather(s2_v, [oi])
            att_v[pl.ds(t * 16, 16)] = jnp.maximum(z, 0.2 * z)

        def scale(k, _):
            a = plsc.load_gather(att_v, [lax.broadcast(k, (16,))])
            for j in range(D // 16):
                rows_v[b, k, pl.ds(j * 16, 16)] = (
                    rows_v[b, k, pl.ds(j * 16, 16)] * a
                )
            return 0

        lax.fori_loop(0, K, scale, 0)

    start_idx(0, 0)
    wait_idx(0)
    start_idx(1, 1)
    start_gather(0)

    def step(i, _):
        for b in range(NBUF):
            c = NBUF * i + b
            wait_gather(b)
            process(b)
            start_scatter(b)
            b2 = (b + 2) % NBUF

            @pl.when(c >= 2)
            def _():
                drain_scatter(b2)

            @pl.when(c + 2 < NCHUNK)
            def _():
                start_idx(c + 2, b2)

            b1 = (b + 1) % NBUF
            wait_idx(b1)
            start_gather(b1)
        return 0

    lax.fori_loop(0, (NCHUNK - 1) // NBUF, step, 0)

    wait_gather(0)
    process(0)
    pltpu.sync_copy(rows_v.at[0], acc_sh.at[sidx_v.at[0]], add=True)
    drain_scatter(2)
    drain_scatter(3)

    plsc.subcore_barrier()
    pltpu.sync_copy(
        acc_sh.at[pl.ds(sid * RPT, RPT)],
        out_hbm.at[cid, pl.ds(sid * RPT, RPT)],
    )


@jax.jit
def kernel(x, edge_index, W, b_w, a_w, a_b):
    a1 = a_w[0, :D]
    a2 = a_w[0, D:]
    u = jnp.zeros((D, 8), jnp.float32).at[:, 0].set(W.T @ a1).at[:, 1].set(W.T @ a2)
    bias = (
        jnp.zeros((1, 8), jnp.float32)
        .at[0, 0].set(jnp.dot(b_w, a1) + a_b[0])
        .at[0, 1].set(jnp.dot(b_w, a2))
    )
    npad_e = EPAD - E
    src_p = jnp.concatenate(
        [edge_index[0], jnp.full((npad_e,), N, jnp.int32)])
    obj_p = jnp.concatenate(
        [edge_index[1], jnp.zeros((npad_e,), jnp.int32)])

    zb = 1000
    z = pl.pallas_call(
        _lin_body,
        grid=(N // zb,),
        in_specs=[
            pl.BlockSpec((zb, D), lambda i: (i, 0)),
            pl.BlockSpec((D, 8), lambda i: (0, 0)),
            pl.BlockSpec((1, 8), lambda i: (0, 0)),
        ],
        out_specs=pl.BlockSpec((zb, 8), lambda i: (i, 0)),
        out_shape=jax.ShapeDtypeStruct((N, 8), jnp.float32),
    )(x, u, bias)

    s1 = z[:, 0]
    s2 = z[:, 1]

    mesh = plsc.VectorSubcoreMesh(core_axis_name="c", subcore_axis_name="s")
    sc_edge = pl.kernel(
        _sc_edge_kernel,
        mesh=mesh,
        compiler_params=pltpu.CompilerParams(needs_layout_passes=False),
        out_type=jax.ShapeDtypeStruct((NC, NPAD, D), jnp.float32),
        scratch_types=[
            pltpu.VMEM((TBL,), jnp.float32),
            pltpu.VMEM((TBL,), jnp.float32),
            pltpu.VMEM((NBUF, K), jnp.int32),
            pltpu.VMEM((NBUF, K), jnp.int32),
            pltpu.VMEM((NBUF, K, D), jnp.float32),
            pltpu.VMEM((K,), jnp.float32),
            pltpu.VMEM_SHARED((NPAD, D), jnp.float32),
        ] + [pltpu.SemaphoreType.DMA] * (3 * NBUF),
    )
    partial_acc = sc_edge(x, src_p, obj_p, s1, s2)

    cb = 1000
    out = pl.pallas_call(
        _combine_body,
        grid=(N // cb,),
        in_specs=[
            pl.BlockSpec((cb, D), lambda i: (i, 0)),
            pl.BlockSpec((cb, D), lambda i: (i, 0)),
            pl.BlockSpec((cb, 8), lambda i: (i, 0)),
            pl.BlockSpec((cb, D), lambda i: (i, 0)),
        ],
        out_specs=pl.BlockSpec((cb, D), lambda i: (i, 0)),
        out_shape=jax.ShapeDtypeStruct((N, D), jnp.float32),
    )(partial_acc[0], partial_acc[1], z, x)
    return out


# speedup vs baseline: 9.6521x; 1.2869x over previous
import jax
import jax.numpy as jnp
from jax import lax
from jax.experimental import pallas as pl
from jax.experimental.pallas import tpu as pltpu
from jax.experimental.pallas import tpu_sc as plsc

N = 10000
D = 128
E = 320000

NC = 2
NS = 16
NW = NC * NS
K = 48
NCHUNK = 209
EPW = NCHUNK * K
EPAD = NW * EPW
NBUF = 4
NPAD = 10112
RPT = NPAD // NS
TBL = 10016


def _lin_body(x_ref, u_ref, b_ref, z_ref):
    z_ref[...] = (
        jnp.dot(x_ref[...], u_ref[...], preferred_element_type=jnp.float32)
        + b_ref[...]
    )


def _combine_body(p0_ref, p1_ref, z_ref, x_ref, o_ref):
    z = z_ref[..., 0:1] + z_ref[..., 1:2]
    att = jnp.maximum(z, 0.2 * z)
    o_ref[...] = p0_ref[...] + p1_ref[...] + att * x_ref[...]


def _sc_edge_kernel(
    x_hbm, src_hbm, obj_hbm, s1_hbm, s2_hbm, out_hbm,
    s1_v, s2_v, sidx_v, oidx_v, rows_v, att_v, acc_sh,
    g0, g1, g2, g3, c0, c1, c2, c3, i0, i1, i2, i3,
):
    gsems = [g0, g1, g2, g3]
    ssems = [c0, c1, c2, c3]
    isems = [i0, i1, i2, i3]
    cid = lax.axis_index("c")
    sid = lax.axis_index("s")
    wid = cid * NS + sid
    ebase = wid * EPW

    pltpu.sync_copy(s1_hbm, s1_v.at[pl.ds(0, N)])
    pltpu.sync_copy(s2_hbm, s2_v.at[pl.ds(0, N)])

    def zfill(i, _):
        for j in range(D // 16):
            rows_v[0, i, pl.ds(j * 16, 16)] = jnp.zeros((16,), jnp.float32)
        return 0

    lax.fori_loop(0, K, zfill, 0)
    for r in range(13):
        pltpu.sync_copy(rows_v.at[0], acc_sh.at[pl.ds(sid * RPT + r * K, K)])
    pltpu.sync_copy(
        rows_v.at[0].at[pl.ds(0, 8)], acc_sh.at[pl.ds(sid * RPT + 624, 8)]
    )
    plsc.subcore_barrier()

    def start_idx(c, b):
        pltpu.async_copy(
            src_hbm.at[pl.ds(ebase + c * K, K)], sidx_v.at[b], isems[b]
        )
        pltpu.async_copy(
            obj_hbm.at[pl.ds(ebase + c * K, K)], oidx_v.at[b], isems[b]
        )

    def wait_idx(b):
        pltpu.make_async_copy(
            src_hbm.at[pl.ds(0, K)], sidx_v.at[b], isems[b]
        ).wait()
        pltpu.make_async_copy(
            obj_hbm.at[pl.ds(0, K)], oidx_v.at[b], isems[b]
        ).wait()

    def start_gather(b):
        pltpu.async_copy(x_hbm.at[oidx_v.at[b]], rows_v.at[b], gsems[b])

    def wait_gather(b):
        pltpu.make_async_copy(
            x_hbm.at[oidx_v.at[b]], rows_v.at[b], gsems[b]
        ).wait()

    def start_scatter(b):
        pltpu.async_copy(
            rows_v.at[b], acc_sh.at[sidx_v.at[b]], ssems[b], add=True
        )

    def drain_scatter(b):
        pltpu.make_async_copy(
            rows_v.at[b], acc_sh.at[sidx_v.at[b]], ssems[b]
        ).wait()

    def process(b):
        for t in range(K // 16):
            si = sidx_v[b, pl.ds(t * 16, 16)]
            oi = oidx_v[b, pl.ds(t * 16, 16)]
            z = plsc.load_gather(s1_v, [si]) + plsc.load_gather(s2_v, [oi])
            att_v[pl.ds(t * 16, 16)] = jnp.maximum(z, 0.2 * z)

        def scale(q, _):
            for r in range(4):
                k = q * 4 + r
                a = plsc.load_gather(att_v, [lax.broadcast(k, (16,))])
                for j in range(D // 16):
                    rows_v[b, k, pl.ds(j * 16, 16)] = (
                        rows_v[b, k, pl.ds(j * 16, 16)] * a
                    )
            return 0

        lax.fori_loop(0, K // 4, scale, 0)

    start_idx(0, 0)
    start_idx(1, 1)
    wait_idx(0)
    start_gather(0)

    def step(i, _):
        for b in range(NBUF):
            c = NBUF * i + b
            b1 = (b + 1) % NBUF
            b2 = (b + 2) % NBUF
            wait_gather(b)
            wait_idx(b1)
            start_gather(b1)
            process(b)
            start_scatter(b)

            @pl.when(c >= 2)
            def _():
                drain_scatter(b2)

            @pl.when(c + 2 < NCHUNK)
            def _():
                start_idx(c + 2, b2)

        return 0

    lax.fori_loop(0, (NCHUNK - 1) // NBUF, step, 0)

    wait_gather(0)
    process(0)
    pltpu.sync_copy(rows_v.at[0], acc_sh.at[sidx_v.at[0]], add=True)
    drain_scatter(2)
    drain_scatter(3)

    plsc.subcore_barrier()
    pltpu.sync_copy(
        acc_sh.at[pl.ds(sid * RPT, RPT)],
        out_hbm.at[cid, pl.ds(sid * RPT, RPT)],
    )


@jax.jit
def kernel(x, edge_index, W, b_w, a_w, a_b):
    a1 = a_w[0, :D]
    a2 = a_w[0, D:]
    u = jnp.zeros((D, 8), jnp.float32).at[:, 0].set(W.T @ a1).at[:, 1].set(W.T @ a2)
    bias = (
        jnp.zeros((1, 8), jnp.float32)
        .at[0, 0].set(jnp.dot(b_w, a1) + a_b[0])
        .at[0, 1].set(jnp.dot(b_w, a2))
    )
    npad_e = EPAD - E
    src_p = jnp.concatenate(
        [edge_index[0], jnp.full((npad_e,), N, jnp.int32)])
    obj_p = jnp.concatenate(
        [edge_index[1], jnp.zeros((npad_e,), jnp.int32)])

    zb = 1000
    z = pl.pallas_call(
        _lin_body,
        grid=(N // zb,),
        in_specs=[
            pl.BlockSpec((zb, D), lambda i: (i, 0)),
            pl.BlockSpec((D, 8), lambda i: (0, 0)),
            pl.BlockSpec((1, 8), lambda i: (0, 0)),
        ],
        out_specs=pl.BlockSpec((zb, 8), lambda i: (i, 0)),
        out_shape=jax.ShapeDtypeStruct((N, 8), jnp.float32),
    )(x, u, bias)

    s1 = z[:, 0]
    s2 = z[:, 1]

    mesh = plsc.VectorSubcoreMesh(core_axis_name="c", subcore_axis_name="s")
    sc_edge = pl.kernel(
        _sc_edge_kernel,
        mesh=mesh,
        compiler_params=pltpu.CompilerParams(needs_layout_passes=False),
        out_type=jax.ShapeDtypeStruct((NC, NPAD, D), jnp.float32),
        scratch_types=[
            pltpu.VMEM((TBL,), jnp.float32),
            pltpu.VMEM((TBL,), jnp.float32),
            pltpu.VMEM((NBUF, K), jnp.int32),
            pltpu.VMEM((NBUF, K), jnp.int32),
            pltpu.VMEM((NBUF, K, D), jnp.float32),
            pltpu.VMEM((K,), jnp.float32),
            pltpu.VMEM_SHARED((NPAD, D), jnp.float32),
        ] + [pltpu.SemaphoreType.DMA] * (3 * NBUF),
    )
    partial_acc = sc_edge(x, src_p, obj_p, s1, s2)

    cb = 1000
    out = pl.pallas_call(
        _combine_body,
        grid=(N // cb,),
        in_specs=[
            pl.BlockSpec((cb, D), lambda i: (i, 0)),
            pl.BlockSpec((cb, D), lambda i: (i, 0)),
            pl.BlockSpec((cb, 8), lambda i: (i, 0)),
            pl.BlockSpec((cb, D), lambda i: (i, 0)),
        ],
        out_specs=pl.BlockSpec((cb, D), lambda i: (i, 0)),
        out_shape=jax.ShapeDtypeStruct((N, D), jnp.float32),
    )(partial_acc[0], partial_acc[1], z, x)
    return out


# speedup vs baseline: 12.7588x; 1.3219x over previous
import jax
import jax.numpy as jnp
from jax import lax
from jax.experimental import pallas as pl
from jax.experimental.pallas import tpu as pltpu
from jax.experimental.pallas import tpu_sc as plsc

N = 10000
D = 128
E = 320000

NC = 2
NS = 16
NW = NC * NS
K = 48
NCHUNK = 209
EPW = NCHUNK * K
EPAD = NW * EPW
NBUF = 4
NIB = 8
NPAD = 10112
RPT = NPAD // NS
TBL = 10016


def _lin_body(x_ref, u_ref, b_ref, z_ref):
    z_ref[...] = (
        jnp.dot(x_ref[...], u_ref[...], preferred_element_type=jnp.float32)
        + b_ref[...]
    )


def _combine_body(p0_ref, p1_ref, z_ref, x_ref, o_ref):
    z = z_ref[..., 0:1] + z_ref[..., 1:2]
    att = jnp.maximum(z, 0.2 * z)
    o_ref[...] = p0_ref[...] + p1_ref[...] + att * x_ref[...]


def _sc_edge_kernel(
    x_hbm, src_hbm, obj_hbm, s1_hbm, s2_hbm, out_hbm,
    s1_v, s2_v, sidx_v, oidx_v, rows_v, att_v, acc_sh,
    g0, g1, g2, g3, c0, c1, c2, c3,
    i0, i1, i2, i3, i4, i5, i6, i7,
):
    gsems = [g0, g1, g2, g3]
    ssems = [c0, c1, c2, c3]
    isems = [i0, i1, i2, i3, i4, i5, i6, i7]
    cid = lax.axis_index("c")
    sid = lax.axis_index("s")
    wid = cid * NS + sid
    ebase = wid * EPW

    pltpu.sync_copy(s1_hbm, s1_v.at[pl.ds(0, N)])
    pltpu.sync_copy(s2_hbm, s2_v.at[pl.ds(0, N)])

    def zfill(i, _):
        for j in range(D // 16):
            rows_v[0, i, pl.ds(j * 16, 16)] = jnp.zeros((16,), jnp.float32)
        return 0

    lax.fori_loop(0, K, zfill, 0)
    for r in range(13):
        pltpu.sync_copy(rows_v.at[0], acc_sh.at[pl.ds(sid * RPT + r * K, K)])
    pltpu.sync_copy(
        rows_v.at[0].at[pl.ds(0, 8)], acc_sh.at[pl.ds(sid * RPT + 624, 8)]
    )
    plsc.subcore_barrier()

    def start_idx(c, ib):
        pltpu.async_copy(
            src_hbm.at[pl.ds(ebase + c * K, K)], sidx_v.at[ib], isems[ib]
        )
        pltpu.async_copy(
            obj_hbm.at[pl.ds(ebase + c * K, K)], oidx_v.at[ib], isems[ib]
        )

    def wait_idx(ib):
        pltpu.make_async_copy(
            src_hbm.at[pl.ds(0, K)], sidx_v.at[ib], isems[ib]
        ).wait()
        pltpu.make_async_copy(
            obj_hbm.at[pl.ds(0, K)], oidx_v.at[ib], isems[ib]
        ).wait()

    def start_gather(b, ib):
        pltpu.async_copy(x_hbm.at[oidx_v.at[ib]], rows_v.at[b], gsems[b])

    def wait_gather(b):
        pltpu.make_async_copy(
            x_hbm.at[oidx_v.at[0]], rows_v.at[b], gsems[b]
        ).wait()

    def start_scatter(b, ib):
        pltpu.async_copy(
            rows_v.at[b], acc_sh.at[sidx_v.at[ib]], ssems[b], add=True
        )

    def drain_scatter(b):
        pltpu.make_async_copy(
            rows_v.at[b], acc_sh.at[sidx_v.at[0]], ssems[b]
        ).wait()

    def process(b, ib):
        for t in range(K // 16):
            si = sidx_v[ib, pl.ds(t * 16, 16)]
            oi = oidx_v[ib, pl.ds(t * 16, 16)]
            z = plsc.load_gather(s1_v, [si]) + plsc.load_gather(s2_v, [oi])
            att_v[pl.ds(t * 16, 16)] = jnp.maximum(z, 0.2 * z)

        def scale(q, _):
            for r in range(4):
                k = q * 4 + r
                a = plsc.load_gather(att_v, [lax.broadcast(k, (16,))])
                for j in range(D // 16):
                    rows_v[b, k, pl.ds(j * 16, 16)] = (
                        rows_v[b, k, pl.ds(j * 16, 16)] * a
                    )
            return 0

        lax.fori_loop(0, K // 4, scale, 0)

    for c in range(4):
        start_idx(c, c)
    wait_idx(0)
    start_gather(0, 0)
    wait_idx(1)
    start_gather(1, 1)

    def step(i, _):
        for u in range(NIB):
            c = NIB * i + u
            b = u % NBUF
            ib = u
            b2 = (b + 2) % NBUF
            ib2 = (u + 2) % NIB
            wait_gather(b)

            @pl.when(c >= 2)
            def _():
                drain_scatter(b2)

            @pl.when(c + 2 < NCHUNK)
            def _():
                wait_idx(ib2)
                start_gather(b2, ib2)

            process(b, ib)
            start_scatter(b, ib)

            @pl.when(c + 4 < NCHUNK)
            def _():
                start_idx(c + 4, (u + 4) % NIB)

        return 0

    lax.fori_loop(0, (NCHUNK - 1) // NIB, step, 0)

    wait_gather(0)
    process(0, 0)
    pltpu.sync_copy(rows_v.at[0], acc_sh.at[sidx_v.at[0]], add=True)
    drain_scatter(2)
    drain_scatter(3)

    plsc.subcore_barrier()
    pltpu.sync_copy(
        acc_sh.at[pl.ds(sid * RPT, RPT)],
        out_hbm.at[cid, pl.ds(sid * RPT, RPT)],
    )


@jax.jit
def kernel(x, edge_index, W, b_w, a_w, a_b):
    a1 = a_w[0, :D]
    a2 = a_w[0, D:]
    u = jnp.zeros((D, 8), jnp.float32).at[:, 0].set(W.T @ a1).at[:, 1].set(W.T @ a2)
    bias = (
        jnp.zeros((1, 8), jnp.float32)
        .at[0, 0].set(jnp.dot(b_w, a1) + a_b[0])
        .at[0, 1].set(jnp.dot(b_w, a2))
    )
    npad_e = EPAD - E
    src_p = jnp.concatenate(
        [edge_index[0], jnp.full((npad_e,), N, jnp.int32)])
    obj_p = jnp.concatenate(
        [edge_index[1], jnp.zeros((npad_e,), jnp.int32)])

    zb = 1000
    z = pl.pallas_call(
        _lin_body,
        grid=(N // zb,),
        in_specs=[
            pl.BlockSpec((zb, D), lambda i: (i, 0)),
            pl.BlockSpec((D, 8), lambda i: (0, 0)),
            pl.BlockSpec((1, 8), lambda i: (0, 0)),
        ],
        out_specs=pl.BlockSpec((zb, 8), lambda i: (i, 0)),
        out_shape=jax.ShapeDtypeStruct((N, 8), jnp.float32),
    )(x, u, bias)

    s1 = z[:, 0]
    s2 = z[:, 1]

    mesh = plsc.VectorSubcoreMesh(core_axis_name="c", subcore_axis_name="s")
    sc_edge = pl.kernel(
        _sc_edge_kernel,
        mesh=mesh,
        compiler_params=pltpu.CompilerParams(needs_layout_passes=False),
        out_type=jax.ShapeDtypeStruct((NC, NPAD, D), jnp.float32),
        scratch_types=[
            pltpu.VMEM((TBL,), jnp.float32),
            pltpu.VMEM((TBL,), jnp.float32),
            pltpu.VMEM((NIB, K), jnp.int32),
            pltpu.VMEM((NIB, K), jnp.int32),
            pltpu.VMEM((NBUF, K, D), jnp.float32),
            pltpu.VMEM((K,), jnp.float32),
            pltpu.VMEM_SHARED((NPAD, D), jnp.float32),
        ] + [pltpu.SemaphoreType.DMA] * (2 * NBUF + NIB),
    )
    partial_acc = sc_edge(x, src_p, obj_p, s1, s2)

    cb = 1000
    out = pl.pallas_call(
        _combine_body,
        grid=(N // cb,),
        in_specs=[
            pl.BlockSpec((cb, D), lambda i: (i, 0)),
            pl.BlockSpec((cb, D), lambda i: (i, 0)),
            pl.BlockSpec((cb, 8), lambda i: (i, 0)),
            pl.BlockSpec((cb, D), lambda i: (i, 0)),
        ],
        out_specs=pl.BlockSpec((cb, D), lambda i: (i, 0)),
        out_shape=jax.ShapeDtypeStruct((N, D), jnp.float32),
    )(partial_acc[0], partial_acc[1], z, x)
    return out
